# Initial kernel scaffold; baseline (speedup 1.0000x reference)
import jax
import jax.numpy as jnp
from jax.experimental import pallas as pl


def kernel(x, edge_index, W, b):
    raise NotImplementedError("write your pallas kernel here")


# speedup vs baseline: 32.1909x; 32.1909x over previous
import functools

import jax
import jax.numpy as jnp
from jax import lax
from jax.experimental import pallas as pl
from jax.experimental.pallas import tpu as pltpu
from jax.experimental.pallas import tpu_sc as plsc

N = 10000
NP = 10240
E = 320000
D = 128
NC = 2
NS = 16
NW = NC * NS
EW = E // NW
SEG = NP // NS

R1, C1 = 80, 125
R3, C3 = 125, 80

_mesh = plsc.VectorSubcoreMesh(core_axis_name="c", subcore_axis_name="s")


def _zero_fill(ref, words):
    for k in range(words // 16):
        ref[pl.ds(k * 16, 16)] = jnp.zeros((16,), jnp.float32)


@functools.partial(
    pl.kernel,
    out_type=jax.ShapeDtypeStruct((NC, 2, NP), jnp.float32),
    mesh=_mesh,
    scratch_types=[
        pltpu.VMEM((R1, C1), jnp.int32),
        pltpu.VMEM((R1, C1), jnp.int32),
        pltpu.VMEM((128,), jnp.float32),
        pltpu.VMEM((SEG,), jnp.float32),
        pltpu.VMEM_SHARED((NP,), jnp.float32),
        pltpu.VMEM_SHARED((NP,), jnp.float32),
    ],
)
def _k1(ei_hbm, degp_hbm, src_v, dst_v, ones_v, seg_v, do_sp, di_sp):
    cid = lax.axis_index("c")
    sid = lax.axis_index("s")
    wid = sid * NC + cid

    _zero_fill(seg_v, SEG)
    for k in range(8):
        ones_v[pl.ds(k * 16, 16)] = jnp.ones((16,), jnp.float32)

    pltpu.sync_copy(ei_hbm.at[0, wid], src_v)
    pltpu.sync_copy(ei_hbm.at[1, wid], dst_v)
    pltpu.sync_copy(seg_v, do_sp.at[pl.ds(sid * SEG, SEG)])
    pltpu.sync_copy(seg_v, di_sp.at[pl.ds(sid * SEG, SEG)])
    plsc.subcore_barrier()

    def row(j, _):
        pltpu.sync_copy(ones_v.at[pl.ds(0, C1)], do_sp.at[src_v.at[j]], add=True)
        pltpu.sync_copy(ones_v.at[pl.ds(0, C1)], di_sp.at[dst_v.at[j]], add=True)
        return 0

    lax.fori_loop(0, R1, row, 0)
    plsc.subcore_barrier()

    pltpu.sync_copy(do_sp.at[pl.ds(sid * SEG, SEG)], seg_v)
    pltpu.sync_copy(seg_v, degp_hbm.at[cid, 0, pl.ds(sid * SEG, SEG)])
    pltpu.sync_copy(di_sp.at[pl.ds(sid * SEG, SEG)], seg_v)
    pltpu.sync_copy(seg_v, degp_hbm.at[cid, 1, pl.ds(sid * SEG, SEG)])


def _k2_body(x3_ref, w_ref, degs_ref, h_ref, nd_ref):
    xw = jnp.sum(x3_ref[...] * w_ref[...], axis=-1)
    deg_out = degs_ref[0, 0] + degs_ref[1, 0]
    deg_in = degs_ref[0, 1] + degs_ref[1, 1]
    h_ref[...] = xw * lax.rsqrt(jnp.maximum(deg_out, 1.0))
    nd_ref[...] = lax.rsqrt(jnp.maximum(deg_in, 1.0))


_k2 = pl.pallas_call(
    _k2_body,
    out_shape=(
        jax.ShapeDtypeStruct((625, 16), jnp.float32),
        jax.ShapeDtypeStruct((625, 16), jnp.float32),
    ),
)


@functools.partial(
    pl.kernel,
    out_type=jax.ShapeDtypeStruct((NC, NP), jnp.float32),
    mesh=_mesh,
    compiler_params=pltpu.CompilerParams(needs_layout_passes=False),
    scratch_types=[
        pltpu.VMEM((R3, C3), jnp.int32),
        pltpu.VMEM((R3, C3), jnp.int32),
        pltpu.VMEM((R3, C3), jnp.float32),
        pltpu.VMEM((N,), jnp.float32),
        pltpu.VMEM((SEG,), jnp.float32),
        pltpu.VMEM_SHARED((NP,), jnp.float32),
    ],
)
def _k3(ei_hbm, h_hbm, aggp_hbm, src_v, dst_v, vals_v, h_v, seg_v, agg_sp):
    cid = lax.axis_index("c")
    sid = lax.axis_index("s")
    wid = sid * NC + cid

    _zero_fill(seg_v, SEG)
    pltpu.sync_copy(ei_hbm.at[0, wid], src_v)
    pltpu.sync_copy(ei_hbm.at[1, wid], dst_v)
    pltpu.sync_copy(h_hbm, h_v)
    pltpu.sync_copy(seg_v, agg_sp.at[pl.ds(sid * SEG, SEG)])
    plsc.subcore_barrier()

    def row(j, _):
        for k in range(C3 // 16):
            idx16 = src_v[j, pl.ds(k * 16, 16)]
            vals_v[j, pl.ds(k * 16, 16)] = plsc.load_gather(h_v, [idx16])
        pltpu.sync_copy(vals_v.at[j], agg_sp.at[dst_v.at[j]], add=True)
        return 0

    lax.fori_loop(0, R3, row, 0)
    plsc.subcore_barrier()

    pltpu.sync_copy(agg_sp.at[pl.ds(sid * SEG, SEG)], seg_v)
    pltpu.sync_copy(seg_v, aggp_hbm.at[cid, pl.ds(sid * SEG, SEG)])


def _k4_body(aggs_ref, nd_ref, b_ref, out_ref):
    agg = aggs_ref[0] + aggs_ref[1]
    out_ref[...] = jnp.maximum(agg * nd_ref[...] + b_ref[0, 0], 0.0)


_k4 = pl.pallas_call(
    _k4_body,
    out_shape=jax.ShapeDtypeStruct((625, 16), jnp.float32),
)


def kernel(x, edge_index, W, b):
    ei1 = edge_index.reshape(2, NW, R1, C1)
    ei3 = edge_index.reshape(2, NW, R3, C3)

    degp = _k1(ei1)
    degs = degp[:, :, :N].reshape(2, 2, 625, 16)
    x3 = x.reshape(625, 16, D)
    w3 = W.reshape(1, 1, D)
    h2, nd2 = _k2(x3, w3, degs)

    aggp = _k3(ei3, h2.reshape(N))
    aggs = aggp[:, :N].reshape(2, 625, 16)
    out2 = _k4(aggs, nd2, b.reshape(1, 1))
    return out2.reshape(N, 1)


# speedup vs baseline: 38.5050x; 1.1961x over previous
import functools

import jax
import jax.numpy as jnp
from jax import lax
from jax.experimental import pallas as pl
from jax.experimental.pallas import tpu as pltpu
from jax.experimental.pallas import tpu_sc as plsc

N = 10000
NP = 10240
E = 320000
D = 128
NC = 2
NS = 16
NW = NC * NS
EW = E // NW
SEG = NP // NS
R, C = 125, 80
W = 1

_mesh = plsc.VectorSubcoreMesh(core_axis_name="c", subcore_axis_name="s")
_params = pltpu.CompilerParams(needs_layout_passes=False)


def _zero_fill(ref, words):
    for k in range(words // 16):
        ref[pl.ds(k * 16, 16)] = jnp.zeros((16,), jnp.float32)


@functools.partial(
    pl.kernel,
    out_type=jax.ShapeDtypeStruct((NC, 2, NP), jnp.float32),
    mesh=_mesh,
    compiler_params=_params,
    scratch_types=[
        pltpu.VMEM((R, C), jnp.int32),
        pltpu.VMEM((R, C), jnp.int32),
        pltpu.VMEM((C,), jnp.float32),
        pltpu.VMEM((SEG,), jnp.float32),
        pltpu.VMEM_SHARED((NP,), jnp.float32),
        pltpu.VMEM_SHARED((NP,), jnp.float32),
        pltpu.SemaphoreType.DMA,
    ],
)
def _k1(es_hbm, degp_hbm, src_v, dst_v, ones_v, seg_v, do_sp, di_sp, sem):
    cid = lax.axis_index("c")
    sid = lax.axis_index("s")
    wid = sid * NC + cid

    _zero_fill(seg_v, SEG)
    for k in range(C // 16):
        ones_v[pl.ds(k * 16, 16)] = jnp.ones((16,), jnp.float32)

    pltpu.sync_copy(es_hbm.at[0, wid], src_v)
    pltpu.sync_copy(es_hbm.at[1, wid], dst_v)
    pltpu.sync_copy(seg_v, do_sp.at[pl.ds(sid * SEG, SEG)])
    pltpu.sync_copy(seg_v, di_sp.at[pl.ds(sid * SEG, SEG)])
    plsc.subcore_barrier()

    def row(j, _):
        pltpu.async_copy(ones_v, do_sp.at[src_v.at[j]], sem, add=True)
        pltpu.async_copy(ones_v, di_sp.at[dst_v.at[j]], sem, add=True)

        @pl.when(j >= W)
        def _():
            pltpu.make_async_copy(ones_v, do_sp.at[src_v.at[j - W]],
                                  sem).wait()
            pltpu.make_async_copy(ones_v, di_sp.at[dst_v.at[j - W]],
                                  sem).wait()

        return 0

    lax.fori_loop(0, R, row, 0)

    def tail(j, _):
        pltpu.make_async_copy(ones_v, do_sp.at[src_v.at[j]], sem).wait()
        pltpu.make_async_copy(ones_v, di_sp.at[dst_v.at[j]], sem).wait()
        return 0

    lax.fori_loop(R - W, R, tail, 0)
    plsc.subcore_barrier()

    pltpu.sync_copy(do_sp.at[pl.ds(sid * SEG, SEG)], seg_v)
    pltpu.sync_copy(seg_v, degp_hbm.at[cid, 0, pl.ds(sid * SEG, SEG)])
    pltpu.sync_copy(di_sp.at[pl.ds(sid * SEG, SEG)], seg_v)
    pltpu.sync_copy(seg_v, degp_hbm.at[cid, 1, pl.ds(sid * SEG, SEG)])


def _k2_body(x3_ref, w_ref, degs_ref, h_ref, nd_ref):
    xw = jnp.sum(x3_ref[...] * w_ref[...], axis=-1)
    deg_out = degs_ref[0, 0] + degs_ref[1, 0]
    deg_in = degs_ref[0, 1] + degs_ref[1, 1]
    h_ref[...] = xw * lax.rsqrt(jnp.maximum(deg_out, 1.0))
    nd_ref[...] = lax.rsqrt(jnp.maximum(deg_in, 1.0))


_k2 = pl.pallas_call(
    _k2_body,
    out_shape=(
        jax.ShapeDtypeStruct((625, 16), jnp.float32),
        jax.ShapeDtypeStruct((625, 16), jnp.float32),
    ),
)


@functools.partial(
    pl.kernel,
    out_type=jax.ShapeDtypeStruct((NC, NP), jnp.float32),
    mesh=_mesh,
    compiler_params=_params,
    scratch_types=[
        pltpu.VMEM((R, C), jnp.int32),
        pltpu.VMEM((R, C), jnp.int32),
        pltpu.VMEM((R, C), jnp.float32),
        pltpu.VMEM((N,), jnp.float32),
        pltpu.VMEM((SEG,), jnp.float32),
        pltpu.VMEM_SHARED((NP,), jnp.float32),
        pltpu.SemaphoreType.DMA,
    ],
)
def _k3(es_hbm, h_hbm, aggp_hbm, src_v, dst_v, vals_v, h_v, seg_v, agg_sp,
        sem):
    cid = lax.axis_index("c")
    sid = lax.axis_index("s")
    wid = sid * NC + cid

    _zero_fill(seg_v, SEG)
    pltpu.sync_copy(es_hbm.at[0, wid], src_v)
    pltpu.sync_copy(es_hbm.at[1, wid], dst_v)
    pltpu.sync_copy(h_hbm, h_v)
    pltpu.sync_copy(seg_v, agg_sp.at[pl.ds(sid * SEG, SEG)])
    plsc.subcore_barrier()

    def row(j, _):
        for k in range(C // 16):
            idx16 = src_v[j, pl.ds(k * 16, 16)]
            vals_v[j, pl.ds(k * 16, 16)] = plsc.load_gather(h_v, [idx16])
        pltpu.async_copy(vals_v.at[j], agg_sp.at[dst_v.at[j]], sem, add=True)

        @pl.when(j >= W)
        def _():
            pltpu.make_async_copy(vals_v.at[j - W],
                                  agg_sp.at[dst_v.at[j - W]], sem).wait()

        return 0

    lax.fori_loop(0, R, row, 0)

    def tail(j, _):
        pltpu.make_async_copy(vals_v.at[j], agg_sp.at[dst_v.at[j]],
                              sem).wait()
        return 0

    lax.fori_loop(R - W, R, tail, 0)
    plsc.subcore_barrier()

    pltpu.sync_copy(agg_sp.at[pl.ds(sid * SEG, SEG)], seg_v)
    pltpu.sync_copy(seg_v, aggp_hbm.at[cid, pl.ds(sid * SEG, SEG)])


def _k4_body(aggs_ref, nd_ref, b_ref, out_ref):
    agg = aggs_ref[0] + aggs_ref[1]
    out_ref[...] = jnp.maximum(agg * nd_ref[...] + b_ref[0, 0], 0.0)


_k4 = pl.pallas_call(
    _k4_body,
    out_shape=jax.ShapeDtypeStruct((625, 16), jnp.float32),
)


def kernel(x, edge_index, W, b):
    es = edge_index.reshape(2, NW, R, C)

    degp = _k1(es)
    degs = degp[:, :, :N].reshape(2, 2, 625, 16)
    x3 = x.reshape(625, 16, D)
    w3 = W.reshape(1, 1, D)
    h2, nd2 = _k2(x3, w3, degs)

    aggp = _k3(es, h2.reshape(N))
    aggs = aggp[:, :N].reshape(2, 625, 16)
    out2 = _k4(aggs, nd2, b.reshape(1, 1))
    return out2.reshape(N, 1)


# speedup vs baseline: 38.6535x; 1.0039x over previous
import functools

import jax
import jax.numpy as jnp
from jax import lax
from jax.experimental import pallas as pl
from jax.experimental.pallas import tpu as pltpu
from jax.experimental.pallas import tpu_sc as plsc

N = 10000
NP = 10240
E = 320000
D = 128
NC = 2
NS = 16
NW = NC * NS
EW = E // NW
SEG = NP // NS
R, C = 125, 80
W = 1

_mesh = plsc.VectorSubcoreMesh(core_axis_name="c", subcore_axis_name="s")
_params = pltpu.CompilerParams(needs_layout_passes=False)


def _zero_fill(ref, words):
    for k in range(words // 16):
        ref[pl.ds(k * 16, 16)] = jnp.zeros((16,), jnp.float32)


@functools.partial(
    pl.kernel,
    out_type=[jax.ShapeDtypeStruct((NP,), jnp.float32) for _ in range(4)],
    mesh=_mesh,
    compiler_params=_params,
    scratch_types=[
        pltpu.VMEM((R, C), jnp.int32),
        pltpu.VMEM((R, C), jnp.int32),
        pltpu.VMEM((C,), jnp.float32),
        pltpu.VMEM((SEG,), jnp.float32),
        pltpu.VMEM_SHARED((NP,), jnp.float32),
        pltpu.VMEM_SHARED((NP,), jnp.float32),
        pltpu.SemaphoreType.DMA,
    ],
)
def _k1(es_hbm, do0_hbm, di0_hbm, do1_hbm, di1_hbm,
        src_v, dst_v, ones_v, seg_v, do_sp, di_sp, sem):
    cid = lax.axis_index("c")
    sid = lax.axis_index("s")
    wid = sid * NC + cid

    _zero_fill(seg_v, SEG)
    for k in range(C // 16):
        ones_v[pl.ds(k * 16, 16)] = jnp.ones((16,), jnp.float32)

    pltpu.sync_copy(es_hbm.at[0, wid], src_v)
    pltpu.sync_copy(es_hbm.at[1, wid], dst_v)
    pltpu.sync_copy(seg_v, do_sp.at[pl.ds(sid * SEG, SEG)])
    pltpu.sync_copy(seg_v, di_sp.at[pl.ds(sid * SEG, SEG)])
    plsc.subcore_barrier()

    def row(j, _):
        pltpu.async_copy(ones_v, do_sp.at[src_v.at[j]], sem, add=True)
        pltpu.async_copy(ones_v, di_sp.at[dst_v.at[j]], sem, add=True)

        @pl.when(j >= W)
        def _():
            pltpu.make_async_copy(ones_v, do_sp.at[src_v.at[j - W]],
                                  sem).wait()
            pltpu.make_async_copy(ones_v, di_sp.at[dst_v.at[j - W]],
                                  sem).wait()

        return 0

    lax.fori_loop(0, R, row, 0)

    def tail(j, _):
        pltpu.make_async_copy(ones_v, do_sp.at[src_v.at[j]], sem).wait()
        pltpu.make_async_copy(ones_v, di_sp.at[dst_v.at[j]], sem).wait()
        return 0

    lax.fori_loop(R - W, R, tail, 0)
    plsc.subcore_barrier()

    sl = pl.ds(sid * SEG, SEG)

    @pl.when(cid == 0)
    def _():
        pltpu.sync_copy(do_sp.at[sl], seg_v)
        pltpu.sync_copy(seg_v, do0_hbm.at[sl])
        pltpu.sync_copy(di_sp.at[sl], seg_v)
        pltpu.sync_copy(seg_v, di0_hbm.at[sl])

    @pl.when(cid == 1)
    def _():
        pltpu.sync_copy(do_sp.at[sl], seg_v)
        pltpu.sync_copy(seg_v, do1_hbm.at[sl])
        pltpu.sync_copy(di_sp.at[sl], seg_v)
        pltpu.sync_copy(seg_v, di1_hbm.at[sl])


def _k2_body(x_ref, w_ref, do0_ref, di0_ref, do1_ref, di1_ref,
             h_ref, nd_ref):
    xw = jnp.sum(x_ref[...] * w_ref[...], axis=-1)
    deg_out = do0_ref[pl.ds(0, N)] + do1_ref[pl.ds(0, N)]
    deg_in = di0_ref[pl.ds(0, N)] + di1_ref[pl.ds(0, N)]
    h_ref[...] = xw * lax.rsqrt(jnp.maximum(deg_out, 1.0))
    nd_ref[...] = lax.rsqrt(jnp.maximum(deg_in, 1.0))


_k2 = pl.pallas_call(
    _k2_body,
    out_shape=(
        jax.ShapeDtypeStruct((N,), jnp.float32),
        jax.ShapeDtypeStruct((N,), jnp.float32),
    ),
)


@functools.partial(
    pl.kernel,
    out_type=[jax.ShapeDtypeStruct((NP,), jnp.float32) for _ in range(2)],
    mesh=_mesh,
    compiler_params=_params,
    scratch_types=[
        pltpu.VMEM((R, C), jnp.int32),
        pltpu.VMEM((R, C), jnp.int32),
        pltpu.VMEM((R, C), jnp.float32),
        pltpu.VMEM((N,), jnp.float32),
        pltpu.VMEM((SEG,), jnp.float32),
        pltpu.VMEM_SHARED((NP,), jnp.float32),
        pltpu.SemaphoreType.DMA,
    ],
)
def _k3(es_hbm, h_hbm, a0_hbm, a1_hbm,
        src_v, dst_v, vals_v, h_v, seg_v, agg_sp, sem):
    cid = lax.axis_index("c")
    sid = lax.axis_index("s")
    wid = sid * NC + cid

    _zero_fill(seg_v, SEG)
    pltpu.sync_copy(es_hbm.at[0, wid], src_v)
    pltpu.sync_copy(es_hbm.at[1, wid], dst_v)
    pltpu.sync_copy(h_hbm, h_v)
    pltpu.sync_copy(seg_v, agg_sp.at[pl.ds(sid * SEG, SEG)])
    plsc.subcore_barrier()

    def row(j, _):
        for k in range(C // 16):
            idx16 = src_v[j, pl.ds(k * 16, 16)]
            vals_v[j, pl.ds(k * 16, 16)] = plsc.load_gather(h_v, [idx16])
        pltpu.async_copy(vals_v.at[j], agg_sp.at[dst_v.at[j]], sem, add=True)

        @pl.when(j >= W)
        def _():
            pltpu.make_async_copy(vals_v.at[j - W],
                                  agg_sp.at[dst_v.at[j - W]], sem).wait()

        return 0

    lax.fori_loop(0, R, row, 0)

    def tail(j, _):
        pltpu.make_async_copy(vals_v.at[j], agg_sp.at[dst_v.at[j]],
                              sem).wait()
        return 0

    lax.fori_loop(R - W, R, tail, 0)
    plsc.subcore_barrier()

    sl = pl.ds(sid * SEG, SEG)
    pltpu.sync_copy(agg_sp.at[sl], seg_v)

    @pl.when(cid == 0)
    def _():
        pltpu.sync_copy(seg_v, a0_hbm.at[sl])

    @pl.when(cid == 1)
    def _():
        pltpu.sync_copy(seg_v, a1_hbm.at[sl])


def _k4_body(a0_ref, a1_ref, nd_ref, b_ref, out_ref):
    agg = a0_ref[pl.ds(0, N)] + a1_ref[pl.ds(0, N)]
    o = jnp.maximum(agg * nd_ref[...] + b_ref[0, 0], 0.0)
    out_ref[...] = o.reshape(N, 1)


_k4 = pl.pallas_call(
    _k4_body,
    out_shape=jax.ShapeDtypeStruct((N, 1), jnp.float32),
)


def kernel(x, edge_index, W_mat, b):
    es = edge_index.reshape(2, NW, R, C)
    wr = W_mat.reshape(1, D)

    do0, di0, do1, di1 = _k1(es)
    h, nd = _k2(x, wr, do0, di0, do1, di1)

    a0, a1 = _k3(es, h)
    return _k4(a0, a1, nd, b.reshape(1, 1))


# speedup vs baseline: 39.4183x; 1.0198x over previous
import functools

import jax
import jax.numpy as jnp
from jax import lax
from jax.experimental import pallas as pl
from jax.experimental.pallas import tpu as pltpu
from jax.experimental.pallas import tpu_sc as plsc

N = 10000
NP = 10240
E = 320000
D = 128
NC = 2
NS = 16
NW = NC * NS
EW = E // NW
SEG = NP // NS
R, C = 125, 80
P = 4

_mesh = plsc.VectorSubcoreMesh(core_axis_name="c", subcore_axis_name="s")
_params = pltpu.CompilerParams(needs_layout_passes=False)


def _zero_fill(ref, words):
    for k in range(words // 16):
        ref[pl.ds(k * 16, 16)] = jnp.zeros((16,), jnp.float32)


def _acc_seg(seg_v, tmp_v, parts, sl):
    pltpu.sync_copy(parts[0].at[sl], seg_v)
    for p in range(1, P):
        pltpu.sync_copy(parts[p].at[sl], tmp_v)
        for k in range(SEG // 16):
            s = pl.ds(k * 16, 16)
            seg_v[s] = seg_v[s] + tmp_v[s]


@functools.partial(
    pl.kernel,
    out_type=[jax.ShapeDtypeStruct((NP,), jnp.float32) for _ in range(4)],
    mesh=_mesh,
    compiler_params=_params,
    scratch_types=[
        pltpu.VMEM((R, C), jnp.int32),
        pltpu.VMEM((R, C), jnp.int32),
        pltpu.VMEM((C,), jnp.float32),
        pltpu.VMEM((SEG,), jnp.float32),
        pltpu.VMEM((SEG,), jnp.float32),
    ]
    + [pltpu.VMEM_SHARED((NP,), jnp.float32) for _ in range(2 * P)]
    + [pltpu.SemaphoreType.DMA],
)
def _k1(es_hbm, do0_hbm, di0_hbm, do1_hbm, di1_hbm,
        src_v, dst_v, ones_v, seg_v, tmp_v, *rest):
    do_sp = rest[:P]
    di_sp = rest[P:2 * P]
    sem = rest[2 * P]
    cid = lax.axis_index("c")
    sid = lax.axis_index("s")
    wid = sid * NC + cid

    _zero_fill(seg_v, SEG)
    for k in range(C // 16):
        ones_v[pl.ds(k * 16, 16)] = jnp.ones((16,), jnp.float32)

    pltpu.sync_copy(es_hbm.at[0, wid], src_v)
    pltpu.sync_copy(es_hbm.at[1, wid], dst_v)
    for p in range(P):
        pltpu.sync_copy(seg_v, do_sp[p].at[pl.ds(sid * SEG, SEG)])
        pltpu.sync_copy(seg_v, di_sp[p].at[pl.ds(sid * SEG, SEG)])
    plsc.subcore_barrier()

    def row(j, _):
        par = lax.rem(j, P)
        for p in range(P):
            @pl.when(par == p)
            def _(p=p):
                pltpu.async_copy(ones_v, do_sp[p].at[src_v.at[j]], sem,
                                 add=True)
                pltpu.async_copy(ones_v, di_sp[p].at[dst_v.at[j]], sem,
                                 add=True)

        @pl.when(j >= P)
        def _():
            jk = j - P
            for p in range(P):
                @pl.when(par == p)
                def _(p=p):
                    pltpu.make_async_copy(ones_v, do_sp[p].at[src_v.at[jk]],
                                          sem).wait()
                    pltpu.make_async_copy(ones_v, di_sp[p].at[dst_v.at[jk]],
                                          sem).wait()

        return 0

    lax.fori_loop(0, R, row, 0)

    def tail(j, _):
        par = lax.rem(j, P)
        for p in range(P):
            @pl.when(par == p)
            def _(p=p):
                pltpu.make_async_copy(ones_v, do_sp[p].at[src_v.at[j]],
                                      sem).wait()
                pltpu.make_async_copy(ones_v, di_sp[p].at[dst_v.at[j]],
                                      sem).wait()
        return 0

    lax.fori_loop(R - P, R, tail, 0)
    plsc.subcore_barrier()

    sl = pl.ds(sid * SEG, SEG)

    @pl.when(cid == 0)
    def _():
        _acc_seg(seg_v, tmp_v, do_sp, sl)
        pltpu.sync_copy(seg_v, do0_hbm.at[sl])
        _acc_seg(seg_v, tmp_v, di_sp, sl)
        pltpu.sync_copy(seg_v, di0_hbm.at[sl])

    @pl.when(cid == 1)
    def _():
        _acc_seg(seg_v, tmp_v, do_sp, sl)
        pltpu.sync_copy(seg_v, do1_hbm.at[sl])
        _acc_seg(seg_v, tmp_v, di_sp, sl)
        pltpu.sync_copy(seg_v, di1_hbm.at[sl])


def _k2_body(x_ref, w_ref, do0_ref, di0_ref, do1_ref, di1_ref,
             h_ref, nd_ref):
    xw = jnp.sum(x_ref[...] * w_ref[...], axis=-1)
    deg_out = do0_ref[pl.ds(0, N)] + do1_ref[pl.ds(0, N)]
    deg_in = di0_ref[pl.ds(0, N)] + di1_ref[pl.ds(0, N)]
    h_ref[...] = xw * lax.rsqrt(jnp.maximum(deg_out, 1.0))
    nd_ref[...] = lax.rsqrt(jnp.maximum(deg_in, 1.0))


_k2 = pl.pallas_call(
    _k2_body,
    out_shape=(
        jax.ShapeDtypeStruct((N,), jnp.float32),
        jax.ShapeDtypeStruct((N,), jnp.float32),
    ),
)


@functools.partial(
    pl.kernel,
    out_type=[jax.ShapeDtypeStruct((NP,), jnp.float32) for _ in range(2)],
    mesh=_mesh,
    compiler_params=_params,
    scratch_types=[
        pltpu.VMEM((R, C), jnp.int32),
        pltpu.VMEM((R, C), jnp.int32),
        pltpu.VMEM((R, C), jnp.float32),
        pltpu.VMEM((N,), jnp.float32),
        pltpu.VMEM((SEG,), jnp.float32),
        pltpu.VMEM((SEG,), jnp.float32),
    ]
    + [pltpu.VMEM_SHARED((NP,), jnp.float32) for _ in range(P)]
    + [pltpu.SemaphoreType.DMA],
)
def _k3(es_hbm, h_hbm, a0_hbm, a1_hbm,
        src_v, dst_v, vals_v, h_v, seg_v, tmp_v, *rest):
    agg_sp = rest[:P]
    sem = rest[P]
    cid = lax.axis_index("c")
    sid = lax.axis_index("s")
    wid = sid * NC + cid

    _zero_fill(seg_v, SEG)
    pltpu.sync_copy(es_hbm.at[0, wid], src_v)
    pltpu.sync_copy(es_hbm.at[1, wid], dst_v)
    pltpu.sync_copy(h_hbm, h_v)
    for p in range(P):
        pltpu.sync_copy(seg_v, agg_sp[p].at[pl.ds(sid * SEG, SEG)])
    plsc.subcore_barrier()

    def row(j, _):
        for k in range(C // 16):
            idx16 = src_v[j, pl.ds(k * 16, 16)]
            vals_v[j, pl.ds(k * 16, 16)] = plsc.load_gather(h_v, [idx16])
        par = lax.rem(j, P)
        for p in range(P):
            @pl.when(par == p)
            def _(p=p):
                pltpu.async_copy(vals_v.at[j], agg_sp[p].at[dst_v.at[j]],
                                 sem, add=True)

        @pl.when(j >= P)
        def _():
            jk = j - P
            for p in range(P):
                @pl.when(par == p)
                def _(p=p):
                    pltpu.make_async_copy(vals_v.at[jk],
                                          agg_sp[p].at[dst_v.at[jk]],
                                          sem).wait()

        return 0

    lax.fori_loop(0, R, row, 0)

    def tail(j, _):
        par = lax.rem(j, P)
        for p in range(P):
            @pl.when(par == p)
            def _(p=p):
                pltpu.make_async_copy(vals_v.at[j],
                                      agg_sp[p].at[dst_v.at[j]], sem).wait()
        return 0

    lax.fori_loop(R - P, R, tail, 0)
    plsc.subcore_barrier()

    sl = pl.ds(sid * SEG, SEG)
    _acc_seg(seg_v, tmp_v, agg_sp, sl)

    @pl.when(cid == 0)
    def _():
        pltpu.sync_copy(seg_v, a0_hbm.at[sl])

    @pl.when(cid == 1)
    def _():
        pltpu.sync_copy(seg_v, a1_hbm.at[sl])


def _k4_body(a0_ref, a1_ref, nd_ref, b_ref, out_ref):
    agg = a0_ref[pl.ds(0, N)] + a1_ref[pl.ds(0, N)]
    o = jnp.maximum(agg * nd_ref[...] + b_ref[0, 0], 0.0)
    out_ref[...] = o.reshape(N, 1)


_k4 = pl.pallas_call(
    _k4_body,
    out_shape=jax.ShapeDtypeStruct((N, 1), jnp.float32),
)


def kernel(x, edge_index, W_mat, b):
    es = edge_index.reshape(2, NW, R, C)
    wr = W_mat.reshape(1, D)

    do0, di0, do1, di1 = _k1(es)
    h, nd = _k2(x, wr, do0, di0, do1, di1)

    a0, a1 = _k3(es, h)
    return _k4(a0, a1, nd, b.reshape(1, 1))
